# Initial kernel scaffold; baseline (speedup 1.0000x reference)
import jax
import jax.numpy as jnp
from jax.experimental import pallas as pl


def kernel(x, edge_index, edge_attr, batch, W1a, b1a, g1, be1, W1b, b1b, g2, be2, Wc0, bc0, Wc1, bc1, W2a, b2a, g3, be3, W2b, b2b):
    raise NotImplementedError("write your pallas kernel here")


# speedup vs baseline: 12.6128x; 12.6128x over previous
import functools

import jax
import jax.numpy as jnp
from jax import lax
from jax.experimental import pallas as pl
from jax.experimental.pallas import tpu as pltpu
from jax.experimental.pallas import tpu_sc as plsc

N = 10000
E = 320000
F = 128
G = 16
EPS = 1e-5

NC = 2
NS = 16
NW = NC * NS
C = 80
EPW = E // NW
NCH = EPW // C
RPT = 624
RTAIL = N - NS * RPT

@functools.cache
def _mesh():
    return plsc.VectorSubcoreMesh(core_axis_name="c", subcore_axis_name="s",
                                  num_cores=NC, num_subcores=NS)


def _deg_body(dst_hbm, ones_hbm, zeros_hbm, out0, out1, ones_v, idx_v, deg_sh):
    c = lax.axis_index("c")
    s = lax.axis_index("s")
    wid = s * NC + c

    @pl.when(s == 0)
    def _zero():
        pltpu.sync_copy(zeros_hbm, deg_sh)

    pltpu.sync_copy(ones_hbm, ones_v)
    plsc.subcore_barrier()

    def chunk(ch, carry):
        off = wid * EPW + ch * C
        pltpu.sync_copy(dst_hbm.at[pl.ds(off, C)], idx_v)
        pltpu.sync_copy(ones_v, deg_sh.at[idx_v], add=True)
        return carry

    lax.fori_loop(0, NCH, chunk, 0)
    plsc.subcore_barrier()

    @pl.when(jnp.logical_and(s == 0, c == 0))
    def _wb0():
        pltpu.sync_copy(deg_sh, out0)

    @pl.when(jnp.logical_and(s == 0, c == 1))
    def _wb1():
        pltpu.sync_copy(deg_sh, out1)


@functools.cache
def _deg_call():
    return pl.kernel(
        _deg_body,
        out_type=(jax.ShapeDtypeStruct((N,), jnp.float32),
                  jax.ShapeDtypeStruct((N,), jnp.float32)),
        mesh=_mesh(),
        scratch_types=[
            pltpu.VMEM((C,), jnp.float32),
            pltpu.VMEM((C,), jnp.int32),
            pltpu.VMEM_SHARED((N,), jnp.float32),
        ],
    )


def _conv_body(g_hbm, src_hbm, dst_hbm, zeros_hbm, out_hbm,
               sidx, didx, rows, acc_sh, sem):
    c = lax.axis_index("c")
    s = lax.axis_index("s")
    wid = s * NC + c

    pltpu.sync_copy(zeros_hbm.at[pl.ds(s * RPT, RPT)],
                    acc_sh.at[pl.ds(s * RPT, RPT)])

    @pl.when(s == NS - 1)
    def _zero_tail():
        pltpu.sync_copy(zeros_hbm.at[pl.ds(NS * RPT, RTAIL)],
                        acc_sh.at[pl.ds(NS * RPT, RTAIL)])

    plsc.subcore_barrier()

    def chunk(ch, carry):
        off = wid * EPW + ch * C
        pltpu.sync_copy(src_hbm.at[pl.ds(off, C)], sidx)
        pltpu.sync_copy(dst_hbm.at[pl.ds(off, C)], didx)
        pltpu.async_copy(g_hbm.at[sidx], rows, sem).wait()
        pltpu.sync_copy(rows, acc_sh.at[didx], add=True)
        return carry

    lax.fori_loop(0, NCH, chunk, 0)
    plsc.subcore_barrier()
    pltpu.sync_copy(acc_sh.at[pl.ds(s * RPT, RPT)],
                    out_hbm.at[c, pl.ds(s * RPT, RPT)])

    @pl.when(s == NS - 1)
    def _wb_tail():
        pltpu.sync_copy(acc_sh.at[pl.ds(NS * RPT, RTAIL)],
                        out_hbm.at[c, pl.ds(NS * RPT, RTAIL)])


@functools.cache
def _conv_call():
    return pl.kernel(
        _conv_body,
        out_type=jax.ShapeDtypeStruct((NC, N, F), jnp.float32),
        mesh=_mesh(),
        scratch_types=[
            pltpu.VMEM((C,), jnp.int32),
            pltpu.VMEM((C,), jnp.int32),
            pltpu.VMEM((C, F), jnp.float32),
            pltpu.VMEM_SHARED((N, F), jnp.float32),
            pltpu.SemaphoreType.DMA,
        ],
    )


def _bn_relu(h, gamma, beta):
    mean = jnp.mean(h, axis=0, keepdims=True)
    var = jnp.mean((h - mean) ** 2, axis=0, keepdims=True)
    return jnp.maximum(gamma * (h - mean) / jnp.sqrt(var + EPS) + beta, 0.0)


def _dinv(dp0, dp1):
    return lax.rsqrt(dp0 + dp1 + 1.0)


def _front_body(x, W1a, b1a, g1, be1, W1b, b1b, g2, be2, Wc0, dp0, dp1, g_out):
    h = jnp.dot(x[...], W1a[...], preferred_element_type=jnp.float32) + b1a[...]
    h = _bn_relu(h, g1[...], be1[...])
    h = jnp.dot(h, W1b[...], preferred_element_type=jnp.float32) + b1b[...]
    h = _bn_relu(h, g2[...], be2[...])
    h2 = jnp.dot(h, Wc0[...], preferred_element_type=jnp.float32)
    g_out[...] = h2 * _dinv(dp0[...], dp1[...])


def _mid_body(acc, g, dp0, dp1, bc0, Wc1, g2_out):
    dinv = _dinv(dp0[...], dp1[...])
    a = acc[...]
    h = jnp.maximum(dinv * (a[0] + a[1] + g[...]) + bc0[...], 0.0)
    h2 = jnp.dot(h, Wc1[...], preferred_element_type=jnp.float32)
    g2_out[...] = h2 * dinv


def _final_body(acc, g, dp0, dp1, bc1, batch, W2a, b2a, g3, be3, W2b, b2b,
                out):
    dinv = _dinv(dp0[...], dp1[...])
    a = acc[...]
    h = jnp.maximum(dinv * (a[0] + a[1] + g[...]) + bc1[...], 0.0)
    b = batch[...]

    row_ids = lax.broadcasted_iota(jnp.int32, (G, 1), 0)

    def seg_row(seg, acc_p):
        row = jnp.max(jnp.where(b == seg, h, -jnp.inf), axis=0, keepdims=True)
        return jnp.where(row_ids == seg, jnp.broadcast_to(row, (G, F)), acc_p)

    pooled = lax.fori_loop(0, G, seg_row, jnp.full((G, F), -jnp.inf,
                                                   jnp.float32))
    p = jnp.dot(pooled, W2a[...], preferred_element_type=jnp.float32) + b2a[...]
    p = _bn_relu(p, g3[...], be3[...])
    out[...] = jnp.dot(p, W2b[...], preferred_element_type=jnp.float32) + b2b[...]


def _tc_call(body, out_shape):
    return pl.pallas_call(body, out_shape=out_shape)


_front_call = _tc_call(_front_body, jax.ShapeDtypeStruct((N, F), jnp.float32))
_mid_call = _tc_call(_mid_body, jax.ShapeDtypeStruct((N, F), jnp.float32))
_final_call = _tc_call(_final_body, jax.ShapeDtypeStruct((G, 1), jnp.float32))


def kernel(x, edge_index, edge_attr, batch, W1a, b1a, g1, be1, W1b, b1b, g2,
           be2, Wc0, bc0, Wc1, bc1, W2a, b2a, g3, be3, W2b, b2b):
    del edge_attr
    src = edge_index[0]
    dst = edge_index[1]
    ones_c = jnp.ones((C,), jnp.float32)
    zeros_n = jnp.zeros((N,), jnp.float32)
    zeros_nf = jnp.zeros((N, F), jnp.float32)

    dp0, dp1 = _deg_call()(dst, ones_c, zeros_n)
    dp0 = dp0.reshape(N, 1)
    dp1 = dp1.reshape(N, 1)

    r = lambda v: v.reshape(1, -1)
    g1v = _front_call(x, W1a, r(b1a), r(g1), r(be1), W1b, r(b1b), r(g2),
                      r(be2), Wc0, dp0, dp1)
    acc1 = _conv_call()(g1v, src, dst, zeros_nf)
    g2v = _mid_call(acc1, g1v, dp0, dp1, r(bc0), Wc1)
    acc2 = _conv_call()(g2v, src, dst, zeros_nf)
    out = _final_call(acc2, g2v, dp0, dp1, r(bc1), batch.reshape(N, 1),
                      W2a, r(b2a), r(g3), r(be3), W2b, r(b2b))
    return out.reshape(G)


# speedup vs baseline: 20.5663x; 1.6306x over previous
import functools

import jax
import jax.numpy as jnp
from jax import lax
from jax.experimental import pallas as pl
from jax.experimental.pallas import tpu as pltpu
from jax.experimental.pallas import tpu_sc as plsc

N = 10000
E = 320000
F = 128
G = 16
EPS = 1e-5

NC = 2
NS = 16
NW = NC * NS
C = 80
EPW = E // NW
NCH = EPW // C
RPT = 624
RTAIL = N - NS * RPT

@functools.cache
def _mesh():
    return plsc.VectorSubcoreMesh(core_axis_name="c", subcore_axis_name="s",
                                  num_cores=NC, num_subcores=NS)


def _deg_body(dst_hbm, ones_hbm, zeros_hbm, out0, out1, ones_v, idx_v, deg_sh):
    c = lax.axis_index("c")
    s = lax.axis_index("s")
    wid = s * NC + c

    @pl.when(s == 0)
    def _zero():
        pltpu.sync_copy(zeros_hbm, deg_sh)

    pltpu.sync_copy(ones_hbm, ones_v)
    plsc.subcore_barrier()

    def chunk(ch, carry):
        off = wid * EPW + ch * C
        pltpu.sync_copy(dst_hbm.at[pl.ds(off, C)], idx_v)
        pltpu.sync_copy(ones_v, deg_sh.at[idx_v], add=True)
        return carry

    lax.fori_loop(0, NCH, chunk, 0)
    plsc.subcore_barrier()

    @pl.when(jnp.logical_and(s == 0, c == 0))
    def _wb0():
        pltpu.sync_copy(deg_sh, out0)

    @pl.when(jnp.logical_and(s == 0, c == 1))
    def _wb1():
        pltpu.sync_copy(deg_sh, out1)


@functools.cache
def _deg_call():
    return pl.kernel(
        _deg_body,
        out_type=(jax.ShapeDtypeStruct((N,), jnp.float32),
                  jax.ShapeDtypeStruct((N,), jnp.float32)),
        mesh=_mesh(),
        scratch_types=[
            pltpu.VMEM((C,), jnp.float32),
            pltpu.VMEM((C,), jnp.int32),
            pltpu.VMEM_SHARED((N,), jnp.float32),
        ],
    )


def _conv_body(g_hbm, src_hbm, dst_hbm, zeros_hbm, out_hbm,
               sidx, didx, rows, acc_sh, sem_g, sem_d, sem_s):
    c = lax.axis_index("c")
    s = lax.axis_index("s")
    wid = s * NC + c

    pltpu.sync_copy(zeros_hbm.at[pl.ds(s * RPT, RPT)],
                    acc_sh.at[pl.ds(s * RPT, RPT)])

    @pl.when(s == NS - 1)
    def _zero_tail():
        pltpu.sync_copy(zeros_hbm.at[pl.ds(NS * RPT, RTAIL)],
                        acc_sh.at[pl.ds(NS * RPT, RTAIL)])

    pltpu.sync_copy(src_hbm.at[pl.ds(wid * EPW, EPW)], sidx)
    plsc.subcore_barrier()

    def gather_d(ch, b):
        return pltpu.make_async_copy(
            g_hbm.at[sidx.at[pl.ds(ch * C, C)]], rows.at[b], sem_g.at[b])

    def didx_d(ch, b):
        return pltpu.make_async_copy(
            dst_hbm.at[pl.ds(wid * EPW + ch * C, C)], didx.at[b],
            sem_d.at[b])

    def scatter_start(b):
        pltpu.async_copy(rows.at[b], acc_sh.at[didx.at[b]], sem_s.at[b],
                         add=True)

    def scatter_wait(b):
        pltpu.make_async_copy(rows.at[b], acc_sh.at[didx.at[b]],
                              sem_s.at[b]).wait()

    didx_d(0, 0).start()
    gather_d(0, 0).start()

    def step(ch, b, skip_scatter_wait=False):
        nb = 1 - b
        gather_d(ch, b).wait()
        if not skip_scatter_wait:
            scatter_wait(nb)
        gather_d(ch + 1, nb).start()
        didx_d(ch + 1, nb).start()
        didx_d(ch, b).wait()
        scatter_start(b)

    def pair(k, carry):
        @pl.when(k == 0)
        def _first():
            step(0, 0, skip_scatter_wait=True)

        @pl.when(k > 0)
        def _steady():
            step(2 * k, 0)

        step(2 * k + 1, 1)
        return carry

    assert NCH % 2 == 1
    lax.fori_loop(0, (NCH - 1) // 2, pair, 0)
    gather_d(NCH - 1, 0).wait()
    scatter_wait(1)
    didx_d(NCH - 1, 0).wait()
    scatter_start(0)
    scatter_wait(0)
    plsc.subcore_barrier()
    pltpu.sync_copy(acc_sh.at[pl.ds(s * RPT, RPT)],
                    out_hbm.at[c, pl.ds(s * RPT, RPT)])

    @pl.when(s == NS - 1)
    def _wb_tail():
        pltpu.sync_copy(acc_sh.at[pl.ds(NS * RPT, RTAIL)],
                        out_hbm.at[c, pl.ds(NS * RPT, RTAIL)])


@functools.cache
def _conv_call():
    return pl.kernel(
        _conv_body,
        out_type=jax.ShapeDtypeStruct((NC, N, F), jnp.float32),
        mesh=_mesh(),
        scratch_types=[
            pltpu.VMEM((EPW,), jnp.int32),
            pltpu.VMEM((2, C), jnp.int32),
            pltpu.VMEM((2, C, F), jnp.float32),
            pltpu.VMEM_SHARED((N, F), jnp.float32),
            pltpu.SemaphoreType.DMA((2,)),
            pltpu.SemaphoreType.DMA((2,)),
            pltpu.SemaphoreType.DMA((2,)),
        ],
    )


def _bn_relu(h, gamma, beta):
    mean = jnp.mean(h, axis=0, keepdims=True)
    var = jnp.mean((h - mean) ** 2, axis=0, keepdims=True)
    return jnp.maximum(gamma * (h - mean) / jnp.sqrt(var + EPS) + beta, 0.0)


def _dinv(dp0, dp1):
    return lax.rsqrt(dp0 + dp1 + 1.0)


def _front_body(x, W1a, b1a, g1, be1, W1b, b1b, g2, be2, Wc0, dp0, dp1, g_out):
    h = jnp.dot(x[...], W1a[...], preferred_element_type=jnp.float32) + b1a[...]
    h = _bn_relu(h, g1[...], be1[...])
    h = jnp.dot(h, W1b[...], preferred_element_type=jnp.float32) + b1b[...]
    h = _bn_relu(h, g2[...], be2[...])
    h2 = jnp.dot(h, Wc0[...], preferred_element_type=jnp.float32)
    g_out[...] = h2 * _dinv(dp0[...], dp1[...])


def _mid_body(acc, g, dp0, dp1, bc0, Wc1, g2_out):
    dinv = _dinv(dp0[...], dp1[...])
    a = acc[...]
    h = jnp.maximum(dinv * (a[0] + a[1] + g[...]) + bc0[...], 0.0)
    h2 = jnp.dot(h, Wc1[...], preferred_element_type=jnp.float32)
    g2_out[...] = h2 * dinv


def _final_body(acc, g, dp0, dp1, bc1, batch, W2a, b2a, g3, be3, W2b, b2b,
                out):
    dinv = _dinv(dp0[...], dp1[...])
    a = acc[...]
    h = jnp.maximum(dinv * (a[0] + a[1] + g[...]) + bc1[...], 0.0)
    b = batch[...]

    row_ids = lax.broadcasted_iota(jnp.int32, (G, 1), 0)

    def seg_row(seg, acc_p):
        row = jnp.max(jnp.where(b == seg, h, -jnp.inf), axis=0, keepdims=True)
        return jnp.where(row_ids == seg, jnp.broadcast_to(row, (G, F)), acc_p)

    pooled = lax.fori_loop(0, G, seg_row, jnp.full((G, F), -jnp.inf,
                                                   jnp.float32))
    p = jnp.dot(pooled, W2a[...], preferred_element_type=jnp.float32) + b2a[...]
    p = _bn_relu(p, g3[...], be3[...])
    out[...] = jnp.dot(p, W2b[...], preferred_element_type=jnp.float32) + b2b[...]


def _tc_call(body, out_shape):
    return pl.pallas_call(body, out_shape=out_shape)


_front_call = _tc_call(_front_body, jax.ShapeDtypeStruct((N, F), jnp.float32))
_mid_call = _tc_call(_mid_body, jax.ShapeDtypeStruct((N, F), jnp.float32))
_final_call = _tc_call(_final_body, jax.ShapeDtypeStruct((G, 1), jnp.float32))


def kernel(x, edge_index, edge_attr, batch, W1a, b1a, g1, be1, W1b, b1b, g2,
           be2, Wc0, bc0, Wc1, bc1, W2a, b2a, g3, be3, W2b, b2b):
    del edge_attr
    src = edge_index[0]
    dst = edge_index[1]
    ones_c = jnp.ones((C,), jnp.float32)
    zeros_n = jnp.zeros((N,), jnp.float32)
    zeros_nf = jnp.zeros((N, F), jnp.float32)

    dp0, dp1 = _deg_call()(dst, ones_c, zeros_n)
    dp0 = dp0.reshape(N, 1)
    dp1 = dp1.reshape(N, 1)

    r = lambda v: v.reshape(1, -1)
    g1v = _front_call(x, W1a, r(b1a), r(g1), r(be1), W1b, r(b1b), r(g2),
                      r(be2), Wc0, dp0, dp1)
    acc1 = _conv_call()(g1v, src, dst, zeros_nf)
    g2v = _mid_call(acc1, g1v, dp0, dp1, r(bc0), Wc1)
    acc2 = _conv_call()(g2v, src, dst, zeros_nf)
    out = _final_call(acc2, g2v, dp0, dp1, r(bc1), batch.reshape(N, 1),
                      W2a, r(b2a), r(g3), r(be3), W2b, r(b2b))
    return out.reshape(G)
